# baseline (speedup 1.0000x reference)
import functools

import jax
import jax.numpy as jnp
from jax import lax
from jax.experimental import pallas as pl
from jax.experimental.pallas import tpu as pltpu
from jax.experimental.pallas import tpu_sc as plsc

SEQ_LEN = 200
BATCH = 1024
DIM = 64
B = SEQ_LEN * BATCH
VOCAB = 1000000
NC = 2
NS = 16
NW = NC * NS
BPW = B // NW
C = 64
NCHUNK = BPW // C
NGRP = C // 16
NBUF = 6
NSUPER = NCHUNK // NBUF
NTAIL = NCHUNK - NSUPER * NBUF

_mesh = plsc.VectorSubcoreMesh(core_axis_name="c", subcore_axis_name="s")


@functools.partial(
    pl.kernel,
    mesh=_mesh,
    compiler_params=pltpu.CompilerParams(needs_layout_passes=False),
    out_type=jax.ShapeDtypeStruct((B, DIM), jnp.float32),
    scratch_types=[
        pltpu.VMEM((NCHUNK, C), jnp.int32),
        pltpu.VMEM((NCHUNK, C), jnp.int32),
        pltpu.VMEM((NBUF, C, 128), jnp.float32),
        pltpu.VMEM((NBUF, C, DIM), jnp.float32),
        pltpu.SemaphoreType.DMA,
    ]
    + [pltpu.SemaphoreType.DMA] * (2 * NBUF),
)
def _gather(tv_hbm, bv_hbm, table_hbm, out_hbm, tv_v, bv_v, pairs_v, out_v,
            isem, *sems):
    gsem = sems[:NBUF]
    wsem = sems[NBUF:]
    wid = lax.axis_index("s") * NC + lax.axis_index("c")
    cbase = wid * NCHUNK

    pltpu.async_copy(tv_hbm.at[wid], tv_v, isem).wait()
    pltpu.async_copy(bv_hbm.at[wid], bv_v, isem).wait()

    def fire_gather(g, b):
        pltpu.async_copy(table_hbm.at[tv_v.at[g]], pairs_v.at[b], gsem[b])

    def wait_gather(b):
        pltpu.make_async_copy(
            table_hbm.at[tv_v.at[0]], pairs_v.at[b], gsem[b]
        ).wait()

    def select(g, b):
        def grp_body(j, carry):
            rv = lax.iota(jnp.int32, 16) + j * 16
            off = bv_v[g, pl.ds(j * 16, 16)]
            for c0 in range(0, DIM, 8):
                vals = [
                    plsc.load_gather(pairs_v.at[b], [rv, off + (c0 + k)])
                    for k in range(8)
                ]
                for k in range(8):
                    plsc.store_scatter(
                        out_v.at[b],
                        [rv, jnp.full((16,), c0 + k, jnp.int32)],
                        vals[k],
                    )
            return carry

        lax.fori_loop(0, NGRP, grp_body, 0)

    def fire_wb(g, b):
        off = pl.multiple_of((cbase + g) * C, 64)
        pltpu.async_copy(out_v.at[b], out_hbm.at[pl.ds(off, C)], wsem[b])

    def wait_wb(b):
        pltpu.make_async_copy(
            out_v.at[b], out_hbm.at[pl.ds(0, C)], wsem[b]
        ).wait()

    for b in range(NBUF):
        fire_gather(b, b)

    def body(s, carry):
        base = s * NBUF
        for b in range(NBUF):
            wait_gather(b)
            wait_wb(b)
            select(base + b, b)
            fire_wb(base + b, b)
        for b in range(NBUF):
            fire_gather(base + NBUF + b, b)
        return carry

    for b in range(NBUF):
        pltpu.async_copy(out_hbm.at[pl.ds(0, C)], out_v.at[b], wsem[b])

    lax.fori_loop(0, NSUPER - 1, body, 0)

    base = (NSUPER - 1) * NBUF
    for b in range(NBUF):
        wait_gather(b)
        wait_wb(b)
        select(base + b, b)
        fire_wb(base + b, b)
    for b in range(NTAIL):
        fire_gather(base + NBUF + b, b)
    for b in range(NTAIL):
        wait_gather(b)
        wait_wb(b)
        select(base + NBUF + b, b)
        fire_wb(base + NBUF + b, b)
    for b in range(NTAIL, NBUF):
        wait_wb(b)
    for b in range(NTAIL):
        wait_wb(b)


def kernel(source, W):
    idx = source.reshape(B)
    tv = (idx >> 1).reshape(NW, NCHUNK, C)
    bv = ((idx & 1) << 6).reshape(NW, NCHUNK, C)
    table2 = W.reshape(VOCAB // 2, 2 * DIM)
    out = _gather(tv, bv, table2)
    return out.reshape(SEQ_LEN, BATCH, DIM)


# speedup vs baseline: 1.1308x; 1.1308x over previous
import functools

import jax
import jax.numpy as jnp
from jax import lax
from jax.experimental import pallas as pl
from jax.experimental.pallas import tpu as pltpu
from jax.experimental.pallas import tpu_sc as plsc

SEQ_LEN = 200
BATCH = 1024
DIM = 64
B = SEQ_LEN * BATCH
VOCAB = 1000000
NC = 2
NS = 16
NW = NC * NS
BPW = B // NW
G = 128
NCHUNK = BPW // G
NBUF = 5
NSUPER = NCHUNK // NBUF
BLK = 1024

_mesh = plsc.VectorSubcoreMesh(core_axis_name="c", subcore_axis_name="s")


@functools.partial(
    pl.kernel,
    mesh=_mesh,
    compiler_params=pltpu.CompilerParams(use_tc_tiling_on_sc=False),
    out_type=jax.ShapeDtypeStruct((B, 2 * DIM), jnp.float32),
    scratch_types=[
        pltpu.VMEM((NCHUNK, G), jnp.int32),
        pltpu.VMEM((NBUF, G, 2 * DIM), jnp.float32),
    ]
    + [pltpu.SemaphoreType.DMA] * (2 * NBUF),
)
def _gather_pairs(tv_hbm, table_hbm, out_hbm, idx_v, rows_v, *sems):
    gsem = sems[:NBUF]
    wsem = sems[NBUF:]
    wid = lax.axis_index("s") * NC + lax.axis_index("c")
    chunk0 = wid * NCHUNK

    pltpu.sync_copy(tv_hbm.at[pl.ds(chunk0, NCHUNK)], idx_v)

    def fire_gather(g, b):
        pltpu.async_copy(table_hbm.at[idx_v.at[g]], rows_v.at[b], gsem[b])

    def wait_gather(b):
        pltpu.make_async_copy(
            table_hbm.at[idx_v.at[0]], rows_v.at[b], gsem[b]
        ).wait()

    def fire_wb(g, b):
        pltpu.async_copy(
            rows_v.at[b], out_hbm.at[pl.ds((chunk0 + g) * G, G)], wsem[b]
        )

    def wait_wb(b):
        pltpu.make_async_copy(
            rows_v.at[b], out_hbm.at[pl.ds(0, G)], wsem[b]
        ).wait()

    for b in range(NBUF):
        fire_gather(b, b)

    def body(s, carry):
        base = s * NBUF
        for b in range(NBUF):
            wait_gather(b)
            fire_wb(base + b, b)
        for b in range(NBUF):
            wait_wb(b)
            fire_gather(base + NBUF + b, b)
        return carry

    lax.fori_loop(0, NSUPER - 1, body, 0)

    last = (NSUPER - 1) * NBUF
    for b in range(NBUF):
        wait_gather(b)
        fire_wb(last + b, b)
    for b in range(NBUF):
        wait_wb(b)


def _select_body(idx_ref, pairs_ref, o_ref):
    x = pairs_ref[...]
    b = (idx_ref[...] & 1).reshape(BLK, 1)
    o_ref[...] = jnp.where(b > 0, x[:, DIM:], x[:, :DIM])


_select = pl.pallas_call(
    _select_body,
    grid=(B // BLK,),
    in_specs=[
        pl.BlockSpec((BLK,), lambda i: (i,)),
        pl.BlockSpec((BLK, 2 * DIM), lambda i: (i, 0)),
    ],
    out_specs=pl.BlockSpec((BLK, DIM), lambda i: (i, 0)),
    out_shape=jax.ShapeDtypeStruct((B, DIM), jnp.float32),
)


def kernel(source, W):
    idx = source.reshape(B)
    tv = (idx >> 1).reshape(NW * NCHUNK, G)
    table2 = W.reshape(VOCAB // 2, 2 * DIM)
    pairs = _gather_pairs(tv, table2)
    out = _select(idx, pairs)
    return out.reshape(SEQ_LEN, BATCH, DIM)


# speedup vs baseline: 1.3389x; 1.1841x over previous
import functools

import jax
import jax.numpy as jnp
from jax import lax
from jax.experimental import pallas as pl
from jax.experimental.pallas import tpu as pltpu
from jax.experimental.pallas import tpu_sc as plsc

SEQ_LEN = 200
BATCH = 1024
DIM = 64
B = SEQ_LEN * BATCH
NC = 2
NS = 16
NW = NC * NS
BPW = B // NW
G = 128
NCHUNK = BPW // G
NBUF = 5
NSUPER = NCHUNK // NBUF

_mesh = plsc.VectorSubcoreMesh(core_axis_name="c", subcore_axis_name="s")


@functools.partial(
    pl.kernel,
    mesh=_mesh,
    compiler_params=pltpu.CompilerParams(use_tc_tiling_on_sc=False),
    out_type=jax.ShapeDtypeStruct((B, DIM), jnp.float32),
    scratch_types=[
        pltpu.VMEM((NCHUNK, G), jnp.int32),
        pltpu.VMEM((NBUF, G, DIM), jnp.float32),
    ]
    + [pltpu.SemaphoreType.DMA] * (2 * NBUF),
)
def _gather(idx_hbm, table_hbm, out_hbm, idx_v, rows_v, *sems):
    gsem = sems[:NBUF]
    wsem = sems[NBUF:]
    wid = lax.axis_index("s") * NC + lax.axis_index("c")
    chunk0 = wid * NCHUNK

    pltpu.sync_copy(idx_hbm.at[pl.ds(chunk0, NCHUNK)], idx_v)

    def fire_gather(g, b):
        pltpu.async_copy(table_hbm.at[idx_v.at[g]], rows_v.at[b], gsem[b])

    def wait_gather(b):
        pltpu.make_async_copy(
            table_hbm.at[idx_v.at[0]], rows_v.at[b], gsem[b]
        ).wait()

    def fire_wb(g, b):
        pltpu.async_copy(
            rows_v.at[b], out_hbm.at[pl.ds((chunk0 + g) * G, G)], wsem[b]
        )

    def wait_wb(b):
        pltpu.make_async_copy(
            rows_v.at[b], out_hbm.at[pl.ds(0, G)], wsem[b]
        ).wait()

    for b in range(NBUF):
        fire_gather(b, b)

    def body(s, carry):
        base = s * NBUF
        for b in range(NBUF):
            wait_gather(b)
            fire_wb(base + b, b)
        for b in range(NBUF):
            wait_wb(b)
            fire_gather(base + NBUF + b, b)
        return carry

    lax.fori_loop(0, NSUPER - 1, body, 0)

    last = (NSUPER - 1) * NBUF
    for b in range(NBUF):
        wait_gather(b)
        fire_wb(last + b, b)
    for b in range(NBUF):
        wait_wb(b)


def kernel(source, W):
    idx = source.reshape(NW * NCHUNK, G)
    out = _gather(idx, W)
    return out.reshape(SEQ_LEN, BATCH, DIM)
